# speedup vs baseline: 6.1531x; 6.1531x over previous
import functools

import jax
import jax.numpy as jnp
from jax import lax
from jax.experimental import pallas as pl
from jax.experimental.pallas import tpu as pltpu
from jax.experimental.pallas import tpu_sc as plsc

NUM_EMBEDDINGS = 100000
EMBEDDING_DIM = 128
BATCH = 4096
HIST = 200

_INFO = plsc.get_sparse_core_info()
_NC = _INFO.num_cores
_NS = _INFO.num_subcores
_NW = _NC * _NS

_TOTAL = BATCH * HIST
_CHUNK = 128
_ROWS_PER_W = _TOTAL // _NW
_CHUNKS_PER_W = _ROWS_PER_W // _CHUNK
_IDX_BLOCK = 8
_OUTER = _CHUNKS_PER_W // _IDX_BLOCK


def _make_kernel():
  mesh = plsc.VectorSubcoreMesh(core_axis_name="c", subcore_axis_name="s")

  @functools.partial(
      pl.kernel,
      mesh=mesh,
      out_type=jax.ShapeDtypeStruct((_TOTAL, EMBEDDING_DIM), jnp.float32),
      scratch_types=[
          pltpu.VMEM((_IDX_BLOCK, _CHUNK), jnp.int32),
          pltpu.VMEM((_CHUNK, EMBEDDING_DIM), jnp.float32),
          pltpu.SemaphoreType.DMA,
      ],
  )
  def emb_kernel(idx_hbm, table_hbm, out_hbm, idx_v, rows_v, sem):
    wid = lax.axis_index("s") * _NC + lax.axis_index("c")
    row0 = wid * _CHUNKS_PER_W

    def outer(i, _):
      pltpu.sync_copy(idx_hbm.at[pl.ds(row0 + i * _IDX_BLOCK, _IDX_BLOCK)],
                      idx_v)

      def inner(j, _):
        pltpu.async_copy(table_hbm.at[idx_v.at[j]], rows_v, sem).wait()
        out_base = (row0 + i * _IDX_BLOCK + j) * _CHUNK
        pltpu.sync_copy(rows_v, out_hbm.at[pl.ds(out_base, _CHUNK)])
        return 0

      lax.fori_loop(0, _IDX_BLOCK, inner, 0, unroll=False)
      return 0

    lax.fori_loop(0, _OUTER, outer, 0, unroll=False)

  return emb_kernel


_EMB = _make_kernel()


@jax.jit
def kernel(input_, weight):
  idx2d = input_.reshape(_TOTAL // _CHUNK, _CHUNK)
  out = _EMB(idx2d, weight)
  return out.reshape(BATCH, HIST, EMBEDDING_DIM)


# speedup vs baseline: 7.5637x; 1.2293x over previous
import functools

import jax
import jax.numpy as jnp
from jax import lax
from jax.experimental import pallas as pl
from jax.experimental.pallas import tpu as pltpu
from jax.experimental.pallas import tpu_sc as plsc

NUM_EMBEDDINGS = 100000
EMBEDDING_DIM = 128
BATCH = 4096
HIST = 200

_INFO = plsc.get_sparse_core_info()
_NC = _INFO.num_cores
_NS = _INFO.num_subcores
_NW = _NC * _NS

_TOTAL = BATCH * HIST
_CHUNK = 128
_ROWS_PER_W = _TOTAL // _NW
_CHUNKS_PER_W = _ROWS_PER_W // _CHUNK
_IDX_BLOCK = 8
_OUTER = _CHUNKS_PER_W // _IDX_BLOCK


def _make_kernel():
  mesh = plsc.VectorSubcoreMesh(core_axis_name="c", subcore_axis_name="s")

  @functools.partial(
      pl.kernel,
      mesh=mesh,
      out_type=jax.ShapeDtypeStruct((_TOTAL, EMBEDDING_DIM), jnp.float32),
      scratch_types=[
          pltpu.VMEM((2, _IDX_BLOCK, _CHUNK), jnp.int32),
          pltpu.VMEM((2, _CHUNK, EMBEDDING_DIM), jnp.float32),
          pltpu.SemaphoreType.DMA,
          pltpu.SemaphoreType.DMA,
          pltpu.SemaphoreType.DMA,
      ],
  )
  def emb_kernel(idx_hbm, table_hbm, out_hbm, idx_v, rows_v, isem, gsem,
                 wsem):
    wid = lax.axis_index("s") * _NC + lax.axis_index("c")
    row0 = wid * _CHUNKS_PER_W

    pltpu.async_copy(idx_hbm.at[pl.ds(row0, _IDX_BLOCK)], idx_v.at[0], isem)

    def outer(i, _):
      islot = lax.rem(i, 2)
      pltpu.make_async_copy(idx_hbm.at[pl.ds(row0, _IDX_BLOCK)],
                            idx_v.at[islot], isem).wait()

      @pl.when(i + 1 < _OUTER)
      def _():
        pltpu.async_copy(
            idx_hbm.at[pl.ds(row0 + (i + 1) * _IDX_BLOCK, _IDX_BLOCK)],
            idx_v.at[1 - islot], isem)

      def inner(j, _):
        c = i * _IDX_BLOCK + j
        slot = lax.rem(j, 2)

        @pl.when(c >= 2)
        def _():
          pltpu.make_async_copy(rows_v.at[slot],
                                out_hbm.at[pl.ds(0, _CHUNK)], wsem).wait()

        pltpu.async_copy(table_hbm.at[idx_v.at[islot, j]], rows_v.at[slot],
                         gsem).wait()
        out_base = (row0 + c) * _CHUNK
        pltpu.async_copy(rows_v.at[slot], out_hbm.at[pl.ds(out_base, _CHUNK)],
                         wsem)
        return 0

      lax.fori_loop(0, _IDX_BLOCK, inner, 0, unroll=False)
      return 0

    lax.fori_loop(0, _OUTER, outer, 0, unroll=False)

    pltpu.make_async_copy(rows_v.at[0], out_hbm.at[pl.ds(0, _CHUNK)],
                          wsem).wait()
    pltpu.make_async_copy(rows_v.at[1], out_hbm.at[pl.ds(0, _CHUNK)],
                          wsem).wait()

  return emb_kernel


_EMB = _make_kernel()


@jax.jit
def kernel(input_, weight):
  idx2d = input_.reshape(_TOTAL // _CHUNK, _CHUNK)
  out = _EMB(idx2d, weight)
  return out.reshape(BATCH, HIST, EMBEDDING_DIM)
